# baseline (speedup 1.0000x reference)
import functools

import jax
import jax.numpy as jnp
from jax.experimental import pallas as pl
from jax.experimental.pallas import tpu as pltpu


def _se_kernel(x_ref, w1_ref, b1_ref, w2t_ref, b2_ref, o_ref, acc_ref,
               *, inv_hw):
    s = pl.program_id(1)
    ns = pl.num_programs(1)

    @pl.when(s == 0)
    def _():
        acc_ref[...] = jnp.zeros_like(acc_ref)

    acc_ref[...] += jnp.sum(x_ref[...], axis=1)

    @pl.when(s == ns - 1)
    def _():
        pooled = acc_ref[...] * inv_hw
        h = jax.lax.dot_general(
            pooled, w1_ref[...], (((1,), (1,)), ((), ())),
            preferred_element_type=jnp.float32)
        h = jnp.maximum(h + b1_ref[...], 0.0)
        y = jnp.dot(h, w2t_ref[...], preferred_element_type=jnp.float32)
        o_ref[...] = jax.nn.sigmoid(y + b2_ref[...])


def kernel(x, w1, b1, w2, b2):
    N, C, H, W = x.shape
    HW = H * W
    hid = w1.shape[0]

    x_flat = jnp.transpose(x.astype(jnp.float32), (0, 2, 3, 1)).reshape(N, HW, C)

    w1_2d = w1.astype(jnp.float32)
    w2t = w2.astype(jnp.float32).T
    b1_2d = b1.astype(jnp.float32).reshape(1, hid)
    b2_2d = b2.astype(jnp.float32).reshape(1, C)

    max_elems = 4 * 1024 * 1024
    TN = min(8, N)
    n_pad = -(-N // TN) * TN
    hw_tile = HW
    while TN * hw_tile * C > max_elems and hw_tile % 2 == 0:
        hw_tile //= 2
    hw_pad = -(-HW // hw_tile) * hw_tile

    if n_pad != N or hw_pad != HW:
        x_flat = jnp.pad(x_flat, ((0, n_pad - N), (0, hw_pad - HW), (0, 0)))

    grid = (n_pad // TN, hw_pad // hw_tile)

    out = pl.pallas_call(
        functools.partial(_se_kernel, inv_hw=1.0 / HW),
        out_shape=jax.ShapeDtypeStruct((n_pad, C), jnp.float32),
        grid=grid,
        in_specs=[
            pl.BlockSpec((TN, hw_tile, C), lambda n, s: (n, s, 0)),
            pl.BlockSpec((hid, C), lambda n, s: (0, 0)),
            pl.BlockSpec((1, hid), lambda n, s: (0, 0)),
            pl.BlockSpec((hid, C), lambda n, s: (0, 0)),
            pl.BlockSpec((1, C), lambda n, s: (0, 0)),
        ],
        out_specs=pl.BlockSpec((TN, C), lambda n, s: (n, 0)),
        scratch_shapes=[pltpu.VMEM((TN, C), jnp.float32)],
        compiler_params=pltpu.CompilerParams(
            dimension_semantics=("parallel", "arbitrary"),
            vmem_limit_bytes=64 * 1024 * 1024,
        ),
    )(x_flat, w1_2d, b1_2d, w2t, b2_2d)

    return out[:N].reshape(N, C, 1, 1)


# speedup vs baseline: 1.0072x; 1.0072x over previous
import functools

import jax
import jax.numpy as jnp
from jax.experimental import pallas as pl
from jax.experimental.pallas import tpu as pltpu


def _se_kernel(x_ref, w1_ref, b1_ref, w2t_ref, b2_ref, o_ref, acc_ref,
               *, inv_hw):
    s = pl.program_id(1)
    ns = pl.num_programs(1)

    @pl.when(s == 0)
    def _():
        acc_ref[...] = jnp.zeros_like(acc_ref)

    acc_ref[...] += jnp.sum(x_ref[...], axis=1)

    @pl.when(s == ns - 1)
    def _():
        pooled = acc_ref[...] * inv_hw
        h = jax.lax.dot_general(
            pooled, w1_ref[...], (((1,), (1,)), ((), ())),
            preferred_element_type=jnp.float32)
        h = jnp.maximum(h + b1_ref[...], 0.0)
        y = jnp.dot(h, w2t_ref[...], preferred_element_type=jnp.float32)
        o_ref[...] = jax.nn.sigmoid(y + b2_ref[...])


def kernel(x, w1, b1, w2, b2):
    N, C, H, W = x.shape
    HW = H * W
    hid = w1.shape[0]

    x_flat = jnp.transpose(x.astype(jnp.float32), (0, 2, 3, 1)).reshape(N, HW, C)

    w1_2d = w1.astype(jnp.float32)
    w2t = w2.astype(jnp.float32).T
    b1_2d = b1.astype(jnp.float32).reshape(1, hid)
    b2_2d = b2.astype(jnp.float32).reshape(1, C)

    max_elems = 4 * 1024 * 1024
    TN = min(8, N)
    n_pad = -(-N // TN) * TN
    hw_tile = HW
    while TN * hw_tile * C > max_elems and hw_tile % 2 == 0:
        hw_tile //= 2
    hw_pad = -(-HW // hw_tile) * hw_tile

    if n_pad != N or hw_pad != HW:
        x_flat = jnp.pad(x_flat, ((0, n_pad - N), (0, hw_pad - HW), (0, 0)))

    grid = (n_pad // TN, hw_pad // hw_tile)

    out = pl.pallas_call(
        functools.partial(_se_kernel, inv_hw=1.0 / HW),
        out_shape=jax.ShapeDtypeStruct((n_pad, C), jnp.float32),
        grid=grid,
        in_specs=[
            pl.BlockSpec((TN, hw_tile, C), lambda n, s: (n, s, 0)),
            pl.BlockSpec((hid, C), lambda n, s: (0, 0)),
            pl.BlockSpec((1, hid), lambda n, s: (0, 0)),
            pl.BlockSpec((hid, C), lambda n, s: (0, 0)),
            pl.BlockSpec((1, C), lambda n, s: (0, 0)),
        ],
        out_specs=pl.BlockSpec((TN, C), lambda n, s: (n, 0)),
        scratch_shapes=[pltpu.VMEM((TN, C), jnp.float32)],
        compiler_params=pltpu.CompilerParams(
            dimension_semantics=("parallel", "arbitrary"),
            vmem_limit_bytes=64 * 1024 * 1024,
        ),
    )(x_flat, w1_2d, b1_2d, w2t, b2_2d)

    return jax.lax.broadcast_in_dim(out[:N], (N, C, 1, 1), (0, 1))
